# baseline (speedup 1.0000x reference)
import functools

import jax
import jax.numpy as jnp
from jax.experimental import pallas as pl
from jax.experimental.pallas import tpu as pltpu

K_SELECT = 300
L_REAL = 1203
L_PAD = 1280
N_ROWS = 5000
NT = 5
TN = N_ROWS // NT
NCHUNK = L_PAD // 128


def _matmul_body(logits_ref, wt_ref, p_ref, c_ref):
    x = logits_ref[0]
    s = jax.nn.sigmoid(x)
    probs = jnp.dot(s, wt_ref[...], preferred_element_type=jnp.float32)
    col = jax.lax.broadcasted_iota(jnp.int32, probs.shape, 1)
    probs = jnp.where(col < L_REAL, probs, -1e30)
    p_ref[0] = probs.reshape(TN * NCHUNK, 128)
    c_ref[0, 0] = probs.reshape(TN, NCHUNK, 128).max(axis=2)


def _stage_a(pred_logits, wt_pad):
    B = pred_logits.shape[0]
    return pl.pallas_call(
        _matmul_body,
        grid=(B, NT),
        in_specs=[
            pl.BlockSpec((1, TN, 256), lambda b, i: (b, i, 0)),
            pl.BlockSpec((256, L_PAD), lambda b, i: (0, 0)),
        ],
        out_specs=[
            pl.BlockSpec((1, TN * NCHUNK, 128), lambda b, i: (b, i, 0)),
            pl.BlockSpec((1, 1, TN, NCHUNK), lambda b, i: (b, i, 0, 0)),
        ],
        out_shape=[
            jax.ShapeDtypeStruct((B, N_ROWS * NCHUNK, 128), jnp.float32),
            jax.ShapeDtypeStruct((B, NT, TN, NCHUNK), jnp.float32),
        ],
    )(pred_logits, wt_pad)


def kernel(pred_logits, pred_boxes, target_sizes, image_names, label_positive_map):
    B = pred_logits.shape[0]
    wt_pad = jnp.zeros((256, L_PAD), jnp.float32).at[:, :L_REAL].set(
        label_positive_map.T)
    p2, _c = _stage_a(pred_logits, wt_pad)

    prob = p2.reshape(B, N_ROWS, L_PAD)[:, :, :L_REAL].reshape(B, -1)
    topk_values, topk_indexes = jax.lax.top_k(prob, K_SELECT)
    scores = topk_values
    topk_boxes = topk_indexes // L_REAL
    labels = topk_indexes % L_REAL

    cx, cy, w, h = jnp.split(pred_boxes, 4, axis=-1)
    boxes = jnp.concatenate(
        [cx - 0.5 * w, cy - 0.5 * h, cx + 0.5 * w, cy + 0.5 * h], axis=-1)
    idx4 = jnp.broadcast_to(topk_boxes[:, :, None], (B, K_SELECT, 4))
    boxes = jnp.take_along_axis(boxes, idx4, axis=1)

    text_masks = jax.nn.sigmoid(pred_logits) > 0.0
    idxV = jnp.broadcast_to(topk_boxes[:, :, None], (B, K_SELECT, 256))
    text_masks = jnp.take_along_axis(text_masks, idxV, axis=1)

    ts = target_sizes.astype(jnp.float32)
    scale_fct = jnp.stack([ts[:, 1], ts[:, 0], ts[:, 1], ts[:, 0]], axis=1)
    boxes = boxes * scale_fct[:, None, :]
    return (text_masks, scores, labels, boxes)


# speedup vs baseline: 12.4854x; 12.4854x over previous
import jax
import jax.numpy as jnp
from jax.experimental import pallas as pl
from jax.experimental.pallas import tpu as pltpu

K_SELECT = 300
L_REAL = 1203
L_PAD = 1280
N_ROWS = 5000
NT = 5
TN = N_ROWS // NT
NCHUNK = L_PAD // 128
NC_TOT = N_ROWS * NCHUNK
NC_PAD = 51200
NCAND = 320
NEG = -1e30
BIGI = 2_000_000_000


def _matmul_body(logits_ref, wt_ref, p_ref, c_ref):
    x = logits_ref[0]
    s = jax.nn.sigmoid(x)
    probs = jnp.dot(s, wt_ref[...], preferred_element_type=jnp.float32)
    col = jax.lax.broadcasted_iota(jnp.int32, probs.shape, 1)
    probs = jnp.where(col < L_REAL, probs, NEG)
    p_ref[0] = probs.reshape(TN * NCHUNK, 128)
    c_ref[0, 0] = probs.reshape(TN, NCHUNK, 128).max(axis=2)


def _stage_a(pred_logits, wt_pad):
    B = pred_logits.shape[0]
    return pl.pallas_call(
        _matmul_body,
        grid=(B, NT),
        in_specs=[
            pl.BlockSpec((1, TN, 256), lambda b, i: (b, i, 0)),
            pl.BlockSpec((256, L_PAD), lambda b, i: (0, 0)),
        ],
        out_specs=[
            pl.BlockSpec((1, TN * NCHUNK, 128), lambda b, i: (b, i, 0)),
            pl.BlockSpec((1, 1, TN, NCHUNK), lambda b, i: (b, i, 0, 0)),
        ],
        out_shape=[
            jax.ShapeDtypeStruct((B, NC_TOT, 128), jnp.float32),
            jax.ShapeDtypeStruct((B, NT, TN, NCHUNK), jnp.float32),
        ],
    )(pred_logits, wt_pad)


def _select_body(c2_ref, p2_ref, boxes_ref, scale_ref,
                 scores_ref, labels_ref, nidx_ref, boxes_out_ref,
                 cs_ref, d_ref, sem):
    b = pl.program_id(0)
    cs_ref[...] = c2_ref[0]

    iota400 = jax.lax.broadcasted_iota(jnp.int32, (1, 400), 1)
    iota128 = jax.lax.broadcasted_iota(jnp.int32, (1, 128), 1)
    iota384 = jax.lax.broadcasted_iota(jnp.int32, (1, 384), 1)

    def s1_body(i, carry):
        S, pvec = carry
        gm = jnp.max(S)
        r = jnp.min(jnp.where(S == gm, iota400, BIGI))
        row = cs_ref[pl.ds(r, 1), :]
        l = jnp.min(jnp.where(row == gm, iota128, BIGI))
        p = r * 128 + l
        pltpu.make_async_copy(p2_ref.at[b, p], d_ref.at[i], sem).start()
        newrow = jnp.where(iota128 == l, NEG, row)
        cs_ref[pl.ds(r, 1), :] = newrow
        S = jnp.where(iota400 == r, jnp.max(newrow), S)
        pvec = jnp.where(iota384 == i, p, pvec)
        return S, pvec

    S0 = jnp.max(cs_ref[...], axis=1).reshape(1, 400)
    pvec0 = jnp.zeros((1, 384), jnp.int32)
    _, pvec = jax.lax.fori_loop(0, NCAND, s1_body, (S0, pvec0))

    def drain(i, _):
        pltpu.make_async_copy(p2_ref.at[b, 0], d_ref.at[i], sem).wait()
        return 0

    jax.lax.fori_loop(0, NCAND, drain, 0)

    nn = pvec // NCHUNK
    cc = pvec - nn * NCHUNK
    base = nn * L_REAL + cc * 128

    d_all = d_ref[...]
    dpad = jnp.pad(d_all, ((0, 384 - NCAND), (0, 0)), constant_values=NEG)
    iota_l2 = jax.lax.broadcasted_iota(jnp.int32, (384, 128), 1)
    Vr = jnp.max(dpad, axis=1)
    Fl = jnp.min(jnp.where(dpad == Vr[:, None], iota_l2, BIGI), axis=1)
    V = Vr.reshape(1, 384)
    F = base + Fl.reshape(1, 384)

    def s2_body(t, carry):
        V, F, outv, outf = carry
        gm = jnp.max(V)
        Fm = jnp.where(V == gm, F, BIGI)
        f = jnp.min(Fm)
        rsel = Fm == f
        r = jnp.min(jnp.where(rsel, iota384, BIGI))
        base_r = jnp.min(jnp.where(rsel, base, BIGI))
        outv = jnp.where(iota384 == t, gm, outv)
        outf = jnp.where(iota384 == t, f, outf)
        l = f - base_r
        row = d_ref[pl.ds(r, 1), :]
        newrow = jnp.where(iota128 == l, NEG, row)
        d_ref[pl.ds(r, 1), :] = newrow
        nv = jnp.max(newrow)
        nfl = jnp.min(jnp.where(newrow == nv, iota128, BIGI))
        V = jnp.where(iota384 == r, nv, V)
        F = jnp.where(iota384 == r, base_r + nfl, F)
        return V, F, outv, outf

    outv0 = jnp.full((1, 384), NEG, jnp.float32)
    outf0 = jnp.zeros((1, 384), jnp.int32)
    _, _, outv, outf = jax.lax.fori_loop(0, K_SELECT, s2_body,
                                         (V, F, outv0, outf0))

    scores_ref[0, 0] = outv[0, :K_SELECT]
    flat = outf[0, :K_SELECT]
    n_idx = flat // L_REAL
    labels_ref[0, 0] = flat - n_idx * L_REAL
    nidx_ref[0, 0] = n_idx

    bx = boxes_ref[0]
    cx, cy, w, h = bx[:, 0:1], bx[:, 1:2], bx[:, 2:3], bx[:, 3:4]
    xyxy = jnp.concatenate(
        [cx - 0.5 * w, cy - 0.5 * h, cx + 0.5 * w, cy + 0.5 * h], axis=-1)
    iota_n = jax.lax.broadcasted_iota(jnp.int32, (K_SELECT, N_ROWS), 1)
    oh = jnp.where(n_idx[:, None] == iota_n, 1.0, 0.0)
    sel = jnp.dot(oh, xyxy, preferred_element_type=jnp.float32,
                  precision=jax.lax.Precision.HIGHEST)
    boxes_out_ref[0, 0] = sel * scale_ref[0, 0]


def _stage_b(c2, p2, pred_boxes, scale):
    B = c2.shape[0]
    return pl.pallas_call(
        _select_body,
        grid=(B,),
        in_specs=[
            pl.BlockSpec((1, 400, 128), lambda b: (b, 0, 0)),
            pl.BlockSpec(memory_space=pl.ANY),
            pl.BlockSpec((1, N_ROWS, 4), lambda b: (b, 0, 0)),
            pl.BlockSpec((1, 1, 4), lambda b: (b, 0, 0)),
        ],
        out_specs=[
            pl.BlockSpec((1, 1, K_SELECT), lambda b: (b, 0, 0)),
            pl.BlockSpec((1, 1, K_SELECT), lambda b: (b, 0, 0)),
            pl.BlockSpec((1, 1, K_SELECT), lambda b: (b, 0, 0)),
            pl.BlockSpec((1, 1, K_SELECT, 4), lambda b: (b, 0, 0, 0)),
        ],
        out_shape=[
            jax.ShapeDtypeStruct((B, 1, K_SELECT), jnp.float32),
            jax.ShapeDtypeStruct((B, 1, K_SELECT), jnp.int32),
            jax.ShapeDtypeStruct((B, 1, K_SELECT), jnp.int32),
            jax.ShapeDtypeStruct((B, 1, K_SELECT, 4), jnp.float32),
        ],
        scratch_shapes=[
            pltpu.VMEM((400, 128), jnp.float32),
            pltpu.VMEM((NCAND, 128), jnp.float32),
            pltpu.SemaphoreType.DMA,
        ],
    )(c2, p2, pred_boxes, scale)


def kernel(pred_logits, pred_boxes, target_sizes, image_names, label_positive_map):
    B = pred_logits.shape[0]
    wt_pad = jnp.zeros((256, L_PAD), jnp.float32).at[:, :L_REAL].set(
        label_positive_map.T)
    p2, c = _stage_a(pred_logits, wt_pad)

    c2 = jnp.pad(c.reshape(B, NC_TOT), ((0, 0), (0, NC_PAD - NC_TOT)),
                 constant_values=NEG).reshape(B, 400, 128)
    ts = target_sizes.astype(jnp.float32)
    scale = jnp.stack([ts[:, 1], ts[:, 0], ts[:, 1], ts[:, 0]],
                      axis=1).reshape(B, 1, 4)

    scores, labels, n_idx, boxes = _stage_b(c2, p2, pred_boxes, scale)
    scores = scores.reshape(B, K_SELECT)
    labels = labels.reshape(B, K_SELECT)
    n_idx = n_idx.reshape(B, K_SELECT)
    boxes = boxes.reshape(B, K_SELECT, 4)

    text_masks = jax.nn.sigmoid(pred_logits) > 0.0
    idxV = jnp.broadcast_to(n_idx[:, :, None], (B, K_SELECT, 256))
    text_masks = jnp.take_along_axis(text_masks, idxV, axis=1)

    return (text_masks, scores, labels, boxes)


# speedup vs baseline: 17.6901x; 1.4169x over previous
import jax
import jax.numpy as jnp
from jax.experimental import pallas as pl
from jax.experimental.pallas import tpu as pltpu

K_SELECT = 300
L_REAL = 1203
L_PAD = 1280
N_ROWS = 5000
NT = 5
TN = N_ROWS // NT
NCHUNK = L_PAD // 128
NC_TOT = N_ROWS * NCHUNK
NC_PAD = 51200
NCAND = 320
NB = 4
NEG = -1e30
BIGI = 2_000_000_000


def _matmul_body(logits_ref, wt_ref, p_ref, c_ref):
    x = logits_ref[0]
    s = jax.nn.sigmoid(x)
    probs = jnp.dot(s, wt_ref[...], preferred_element_type=jnp.float32)
    col = jax.lax.broadcasted_iota(jnp.int32, probs.shape, 1)
    probs = jnp.where(col < L_REAL, probs, NEG)
    p_ref[0] = probs.reshape(TN * NCHUNK, 128)
    c_ref[0, 0] = probs.reshape(TN, NCHUNK, 128).max(axis=2)


def _stage_a(pred_logits, wt_pad):
    B = pred_logits.shape[0]
    return pl.pallas_call(
        _matmul_body,
        grid=(B, NT),
        in_specs=[
            pl.BlockSpec((1, TN, 256), lambda b, i: (b, i, 0)),
            pl.BlockSpec((256, L_PAD), lambda b, i: (0, 0)),
        ],
        out_specs=[
            pl.BlockSpec((1, TN * NCHUNK, 128), lambda b, i: (b, i, 0)),
            pl.BlockSpec((1, 1, TN, NCHUNK), lambda b, i: (b, i, 0, 0)),
        ],
        out_shape=[
            jax.ShapeDtypeStruct((B, NC_TOT, 128), jnp.float32),
            jax.ShapeDtypeStruct((B, NT, TN, NCHUNK), jnp.float32),
        ],
    )(pred_logits, wt_pad)


def _select_body(c2_ref, p2_ref, boxes_ref, scale_ref,
                 scores_ref, labels_ref, nidx_ref, boxes_out_ref,
                 cs_ref, d_ref, sem):
    cs_ref[...] = c2_ref[...]

    iota400 = jax.lax.broadcasted_iota(jnp.int32, (NB, 400), 1)
    iota128 = jax.lax.broadcasted_iota(jnp.int32, (1, 128), 1)
    iota384 = jax.lax.broadcasted_iota(jnp.int32, (NB, 384), 1)

    def s1_body(i, carry):
        S, pvec = carry
        gm = jnp.max(S, axis=1, keepdims=True)
        rv = jnp.min(jnp.where(S == gm, iota400, BIGI), axis=1)
        nvs, ps = [], []
        for b in range(NB):
            r = rv[b]
            row = cs_ref[b, pl.ds(r, 1), :]
            l = jnp.min(jnp.where(row == jnp.max(row), iota128, BIGI))
            p = r * 128 + l
            pltpu.make_async_copy(p2_ref.at[b, p], d_ref.at[b, i], sem).start()
            newrow = jnp.where(iota128 == l, NEG, row)
            cs_ref[b, pl.ds(r, 1), :] = newrow
            nvs.append(jnp.max(newrow))
            ps.append(p)
        nvv = jnp.stack(nvs)
        pvv = jnp.stack(ps)
        S = jnp.where(iota400 == rv[:, None], nvv[:, None], S)
        pvec = jnp.where(iota384 == i, pvv[:, None], pvec)
        return S, pvec

    S0 = jnp.max(cs_ref[...], axis=2)
    pvec0 = jnp.zeros((NB, 384), jnp.int32)
    _, pvec = jax.lax.fori_loop(0, NCAND, s1_body, (S0, pvec0))

    def drain(i, _):
        for b in range(NB):
            pltpu.make_async_copy(p2_ref.at[b, 0], d_ref.at[b, i], sem).wait()
        return 0

    jax.lax.fori_loop(0, NCAND, drain, 0)

    nn = pvec // NCHUNK
    cc = pvec - nn * NCHUNK
    base = nn * L_REAL + cc * 128

    d_all = d_ref[...]
    dpad = jnp.pad(d_all, ((0, 0), (0, 384 - NCAND), (0, 0)),
                   constant_values=NEG)
    iota_l3 = jax.lax.broadcasted_iota(jnp.int32, (NB, 384, 128), 2)
    V = jnp.max(dpad, axis=2)
    Fl = jnp.min(jnp.where(dpad == V[:, :, None], iota_l3, BIGI), axis=2)
    F = base + Fl

    def s2_body(t, carry):
        V, F, outv, outf = carry
        gm = jnp.max(V, axis=1, keepdims=True)
        Fm = jnp.where(V == gm, F, BIGI)
        f = jnp.min(Fm, axis=1)
        rsel = Fm == f[:, None]
        rv = jnp.min(jnp.where(rsel, iota384, BIGI), axis=1)
        basev = jnp.min(jnp.where(rsel, base, BIGI), axis=1)
        outv = jnp.where(iota384 == t, gm, outv)
        outf = jnp.where(iota384 == t, f[:, None], outf)
        lv = f - basev
        nvs, nfls = [], []
        for b in range(NB):
            r = rv[b]
            row = d_ref[b, pl.ds(r, 1), :]
            newrow = jnp.where(iota128 == lv[b], NEG, row)
            d_ref[b, pl.ds(r, 1), :] = newrow
            nv = jnp.max(newrow)
            nvs.append(nv)
            nfls.append(jnp.min(jnp.where(newrow == nv, iota128, BIGI)))
        nvv = jnp.stack(nvs)
        nfv = basev + jnp.stack(nfls)
        rmask = iota384 == rv[:, None]
        V = jnp.where(rmask, nvv[:, None], V)
        F = jnp.where(rmask, nfv[:, None], F)
        return V, F, outv, outf

    outv0 = jnp.full((NB, 384), NEG, jnp.float32)
    outf0 = jnp.zeros((NB, 384), jnp.int32)
    _, _, outv, outf = jax.lax.fori_loop(0, K_SELECT, s2_body,
                                         (V, F, outv0, outf0))

    scores_ref[...] = outv[:, None, :K_SELECT]
    flat = outf[:, :K_SELECT]
    n_idx = flat // L_REAL
    labels_ref[...] = (flat - n_idx * L_REAL)[:, None, :]
    nidx_ref[...] = n_idx[:, None, :]

    iota_n = jax.lax.broadcasted_iota(jnp.int32, (K_SELECT, N_ROWS), 1)
    for b in range(NB):
        bx = boxes_ref[b]
        cx, cy, w, h = bx[:, 0:1], bx[:, 1:2], bx[:, 2:3], bx[:, 3:4]
        xyxy = jnp.concatenate(
            [cx - 0.5 * w, cy - 0.5 * h, cx + 0.5 * w, cy + 0.5 * h], axis=-1)
        oh = jnp.where(n_idx[b][:, None] == iota_n, 1.0, 0.0)
        sel = jnp.dot(oh, xyxy, preferred_element_type=jnp.float32,
                      precision=jax.lax.Precision.HIGHEST)
        boxes_out_ref[b, 0] = sel * scale_ref[b, 0]


def _stage_b(c2, p2, pred_boxes, scale):
    B = c2.shape[0]
    return pl.pallas_call(
        _select_body,
        grid=(1,),
        in_specs=[
            pl.BlockSpec((B, 400, 128), lambda i: (0, 0, 0)),
            pl.BlockSpec(memory_space=pl.ANY),
            pl.BlockSpec((B, N_ROWS, 4), lambda i: (0, 0, 0)),
            pl.BlockSpec((B, 1, 4), lambda i: (0, 0, 0)),
        ],
        out_specs=[
            pl.BlockSpec((B, 1, K_SELECT), lambda i: (0, 0, 0)),
            pl.BlockSpec((B, 1, K_SELECT), lambda i: (0, 0, 0)),
            pl.BlockSpec((B, 1, K_SELECT), lambda i: (0, 0, 0)),
            pl.BlockSpec((B, 1, K_SELECT, 4), lambda i: (0, 0, 0, 0)),
        ],
        out_shape=[
            jax.ShapeDtypeStruct((B, 1, K_SELECT), jnp.float32),
            jax.ShapeDtypeStruct((B, 1, K_SELECT), jnp.int32),
            jax.ShapeDtypeStruct((B, 1, K_SELECT), jnp.int32),
            jax.ShapeDtypeStruct((B, 1, K_SELECT, 4), jnp.float32),
        ],
        scratch_shapes=[
            pltpu.VMEM((NB, 400, 128), jnp.float32),
            pltpu.VMEM((NB, NCAND, 128), jnp.float32),
            pltpu.SemaphoreType.DMA,
        ],
    )(c2, p2, pred_boxes, scale)


def kernel(pred_logits, pred_boxes, target_sizes, image_names, label_positive_map):
    B = pred_logits.shape[0]
    wt_pad = jnp.zeros((256, L_PAD), jnp.float32).at[:, :L_REAL].set(
        label_positive_map.T)
    p2, c = _stage_a(pred_logits, wt_pad)

    c2 = jnp.pad(c.reshape(B, NC_TOT), ((0, 0), (0, NC_PAD - NC_TOT)),
                 constant_values=NEG).reshape(B, 400, 128)
    ts = target_sizes.astype(jnp.float32)
    scale = jnp.stack([ts[:, 1], ts[:, 0], ts[:, 1], ts[:, 0]],
                      axis=1).reshape(B, 1, 4)

    scores, labels, n_idx, boxes = _stage_b(c2, p2, pred_boxes, scale)
    scores = scores.reshape(B, K_SELECT)
    labels = labels.reshape(B, K_SELECT)
    n_idx = n_idx.reshape(B, K_SELECT)
    boxes = boxes.reshape(B, K_SELECT, 4)

    text_masks = jax.nn.sigmoid(pred_logits) > 0.0
    idxV = jnp.broadcast_to(n_idx[:, :, None], (B, K_SELECT, 256))
    text_masks = jnp.take_along_axis(text_masks, idxV, axis=1)

    return (text_masks, scores, labels, boxes)


# speedup vs baseline: 27.1719x; 1.5360x over previous
import jax
import jax.numpy as jnp
from jax.experimental import pallas as pl
from jax.experimental.pallas import tpu as pltpu

K_SELECT = 300
L_REAL = 1203
L_PAD = 1280
N_ROWS = 5000
NTILE = 5
TN = N_ROWS // NTILE
NEG = -1e30
BIGI = 2_000_000_000


def _fused_body(logits_ref, wt_ref, boxes_ref, scale_ref,
                scores_ref, labels_ref, nidx_ref, boxes_out_ref, pv_ref):
    col = jax.lax.broadcasted_iota(jnp.int32, (TN, L_PAD), 1)
    v_parts = []
    for t in range(NTILE):
        x = logits_ref[0, t * TN:(t + 1) * TN, :]
        s = jax.nn.sigmoid(x)
        probs = jnp.dot(s, wt_ref[...], preferred_element_type=jnp.float32)
        probs = jnp.where(col < L_REAL, probs, NEG)
        pv_ref[t * TN:(t + 1) * TN, :] = probs
        v_parts.append(jnp.max(probs, axis=1))
    V = jnp.stack(v_parts)

    pos2 = jax.lax.broadcasted_iota(jnp.int32, (NTILE, TN), 0) * TN + \
        jax.lax.broadcasted_iota(jnp.int32, (NTILE, TN), 1)
    iota_l = jax.lax.broadcasted_iota(jnp.int32, (1, L_PAD), 1)
    iota384 = jax.lax.broadcasted_iota(jnp.int32, (1, 384), 1)

    def body(t, carry):
        V, outv, outn, outl = carry
        gm = jnp.max(V)
        r = jnp.min(jnp.where(V == gm, pos2, BIGI))
        row = pv_ref[pl.ds(r, 1), :]
        l = jnp.min(jnp.where(row == gm, iota_l, BIGI))
        outv = jnp.where(iota384 == t, gm, outv)
        outn = jnp.where(iota384 == t, r, outn)
        outl = jnp.where(iota384 == t, l, outl)
        newrow = jnp.where(iota_l == l, NEG, row)
        pv_ref[pl.ds(r, 1), :] = newrow
        V = jnp.where(pos2 == r, jnp.max(newrow), V)
        return V, outv, outn, outl

    outv0 = jnp.full((1, 384), NEG, jnp.float32)
    outi0 = jnp.zeros((1, 384), jnp.int32)
    _, outv, outn, outl = jax.lax.fori_loop(
        0, K_SELECT, body, (V, outv0, outi0, outi0))

    scores_ref[0, 0] = outv[0, :K_SELECT]
    labels_ref[0, 0] = outl[0, :K_SELECT]
    n_idx = outn[0, :K_SELECT]
    nidx_ref[0, 0] = n_idx

    bx = boxes_ref[0]
    cx, cy, w, h = bx[:, 0:1], bx[:, 1:2], bx[:, 2:3], bx[:, 3:4]
    xyxy = jnp.concatenate(
        [cx - 0.5 * w, cy - 0.5 * h, cx + 0.5 * w, cy + 0.5 * h], axis=-1)
    iota_n = jax.lax.broadcasted_iota(jnp.int32, (K_SELECT, N_ROWS), 1)
    oh = jnp.where(n_idx[:, None] == iota_n, 1.0, 0.0)
    sel = jnp.dot(oh, xyxy, preferred_element_type=jnp.float32,
                  precision=jax.lax.Precision.HIGHEST)
    boxes_out_ref[0, 0] = sel * scale_ref[0, 0]


def _fused(pred_logits, wt_pad, pred_boxes, scale):
    B = pred_logits.shape[0]
    return pl.pallas_call(
        _fused_body,
        grid=(B,),
        in_specs=[
            pl.BlockSpec((1, N_ROWS, 256), lambda b: (b, 0, 0)),
            pl.BlockSpec((256, L_PAD), lambda b: (0, 0)),
            pl.BlockSpec((1, N_ROWS, 4), lambda b: (b, 0, 0)),
            pl.BlockSpec((1, 1, 4), lambda b: (b, 0, 0)),
        ],
        out_specs=[
            pl.BlockSpec((1, 1, K_SELECT), lambda b: (b, 0, 0)),
            pl.BlockSpec((1, 1, K_SELECT), lambda b: (b, 0, 0)),
            pl.BlockSpec((1, 1, K_SELECT), lambda b: (b, 0, 0)),
            pl.BlockSpec((1, 1, K_SELECT, 4), lambda b: (b, 0, 0, 0)),
        ],
        out_shape=[
            jax.ShapeDtypeStruct((B, 1, K_SELECT), jnp.float32),
            jax.ShapeDtypeStruct((B, 1, K_SELECT), jnp.int32),
            jax.ShapeDtypeStruct((B, 1, K_SELECT), jnp.int32),
            jax.ShapeDtypeStruct((B, 1, K_SELECT, 4), jnp.float32),
        ],
        scratch_shapes=[
            pltpu.VMEM((N_ROWS, L_PAD), jnp.float32),
        ],
    )(pred_logits, wt_pad, pred_boxes, scale)


def kernel(pred_logits, pred_boxes, target_sizes, image_names, label_positive_map):
    B = pred_logits.shape[0]
    wt_pad = jnp.zeros((256, L_PAD), jnp.float32).at[:, :L_REAL].set(
        label_positive_map.T)
    ts = target_sizes.astype(jnp.float32)
    scale = jnp.stack([ts[:, 1], ts[:, 0], ts[:, 1], ts[:, 0]],
                      axis=1).reshape(B, 1, 4)

    scores, labels, n_idx, boxes = _fused(pred_logits, wt_pad, pred_boxes, scale)
    scores = scores.reshape(B, K_SELECT)
    labels = labels.reshape(B, K_SELECT)
    n_idx = n_idx.reshape(B, K_SELECT)
    boxes = boxes.reshape(B, K_SELECT, 4)

    text_masks = jax.nn.sigmoid(pred_logits) > 0.0
    idxV = jnp.broadcast_to(n_idx[:, :, None], (B, K_SELECT, 256))
    text_masks = jnp.take_along_axis(text_masks, idxV, axis=1)

    return (text_masks, scores, labels, boxes)


# speedup vs baseline: 31.7302x; 1.1678x over previous
import jax
import jax.numpy as jnp
from jax.experimental import pallas as pl
from jax.experimental.pallas import tpu as pltpu

K_SELECT = 300
L_REAL = 1203
L_PAD = 1280
N_ROWS = 5000
NTILE = 5
TN = N_ROWS // NTILE
NEG = -1e30
BIGI = 2_000_000_000


def _fused_body(logits_ref, wt_ref, boxes_ref, scale_ref,
                scores_ref, labels_ref, nidx_ref, boxes_out_ref, pv_ref):
    col = jax.lax.broadcasted_iota(jnp.int32, (TN, L_PAD), 1)
    v_parts = []
    for t in range(NTILE):
        x = logits_ref[0, t * TN:(t + 1) * TN, :]
        s = jax.nn.sigmoid(x)
        probs = jnp.dot(s, wt_ref[...], preferred_element_type=jnp.float32)
        probs = jnp.where(col < L_REAL, probs, NEG)
        pv_ref[t * TN:(t + 1) * TN, :] = probs
        v_parts.append(jnp.max(probs, axis=1))
    V = jnp.stack(v_parts)

    pos2 = jax.lax.broadcasted_iota(jnp.int32, (NTILE, TN), 0) * TN + \
        jax.lax.broadcasted_iota(jnp.int32, (NTILE, TN), 1)
    iota_l = jax.lax.broadcasted_iota(jnp.int32, (1, L_PAD), 1)
    iota384 = jax.lax.broadcasted_iota(jnp.int32, (1, 384), 1)

    def body(t, carry):
        V, gm, r, outv, outn, outl = carry
        row = pv_ref[pl.ds(r, 1), :]
        l = jnp.min(jnp.where(row == gm, iota_l, BIGI))
        outv = jnp.where(iota384 == t, gm, outv)
        outn = jnp.where(iota384 == t, r, outn)
        outl = jnp.where(iota384 == t, l, outl)
        newrow = jnp.where(iota_l == l, NEG, row)
        pv_ref[pl.ds(r, 1), :] = newrow
        nv = jnp.max(newrow)
        vmask = jnp.where(pos2 == r, NEG, V)
        m2 = jnp.max(vmask)
        rA = jnp.min(jnp.where(vmask == m2, pos2, BIGI))
        gm_n = jnp.maximum(nv, m2)
        r_n = jnp.where(nv > m2, r, jnp.where(nv < m2, rA, jnp.minimum(r, rA)))
        V = jnp.where(pos2 == r, nv, V)
        return V, gm_n, r_n, outv, outn, outl

    outv0 = jnp.full((1, 384), NEG, jnp.float32)
    outi0 = jnp.zeros((1, 384), jnp.int32)
    gm0 = jnp.max(V)
    r0 = jnp.min(jnp.where(V == gm0, pos2, BIGI))
    _, _, _, outv, outn, outl = jax.lax.fori_loop(
        0, K_SELECT, body, (V, gm0, r0, outv0, outi0, outi0))

    scores_ref[0, 0] = outv[0, :K_SELECT]
    labels_ref[0, 0] = outl[0, :K_SELECT]
    n_idx = outn[0, :K_SELECT]
    nidx_ref[0, 0] = n_idx

    bx = boxes_ref[0]
    cx, cy, w, h = bx[:, 0:1], bx[:, 1:2], bx[:, 2:3], bx[:, 3:4]
    xyxy = jnp.concatenate(
        [cx - 0.5 * w, cy - 0.5 * h, cx + 0.5 * w, cy + 0.5 * h], axis=-1)
    iota_n = jax.lax.broadcasted_iota(jnp.int32, (K_SELECT, N_ROWS), 1)
    oh = jnp.where(n_idx[:, None] == iota_n, 1.0, 0.0)
    sel = jnp.dot(oh, xyxy, preferred_element_type=jnp.float32,
                  precision=jax.lax.Precision.HIGHEST)
    boxes_out_ref[0, 0] = sel * scale_ref[0, 0]


def _fused(pred_logits, wt_pad, pred_boxes, scale):
    B = pred_logits.shape[0]
    return pl.pallas_call(
        _fused_body,
        grid=(B,),
        in_specs=[
            pl.BlockSpec((1, N_ROWS, 256), lambda b: (b, 0, 0)),
            pl.BlockSpec((256, L_PAD), lambda b: (0, 0)),
            pl.BlockSpec((1, N_ROWS, 4), lambda b: (b, 0, 0)),
            pl.BlockSpec((1, 1, 4), lambda b: (b, 0, 0)),
        ],
        out_specs=[
            pl.BlockSpec((1, 1, K_SELECT), lambda b: (b, 0, 0)),
            pl.BlockSpec((1, 1, K_SELECT), lambda b: (b, 0, 0)),
            pl.BlockSpec((1, 1, K_SELECT), lambda b: (b, 0, 0)),
            pl.BlockSpec((1, 1, K_SELECT, 4), lambda b: (b, 0, 0, 0)),
        ],
        out_shape=[
            jax.ShapeDtypeStruct((B, 1, K_SELECT), jnp.float32),
            jax.ShapeDtypeStruct((B, 1, K_SELECT), jnp.int32),
            jax.ShapeDtypeStruct((B, 1, K_SELECT), jnp.int32),
            jax.ShapeDtypeStruct((B, 1, K_SELECT, 4), jnp.float32),
        ],
        scratch_shapes=[
            pltpu.VMEM((N_ROWS, L_PAD), jnp.float32),
        ],
    )(pred_logits, wt_pad, pred_boxes, scale)


def kernel(pred_logits, pred_boxes, target_sizes, image_names, label_positive_map):
    B = pred_logits.shape[0]
    wt_pad = jnp.zeros((256, L_PAD), jnp.float32).at[:, :L_REAL].set(
        label_positive_map.T)
    ts = target_sizes.astype(jnp.float32)
    scale = jnp.stack([ts[:, 1], ts[:, 0], ts[:, 1], ts[:, 0]],
                      axis=1).reshape(B, 1, 4)

    scores, labels, n_idx, boxes = _fused(pred_logits, wt_pad, pred_boxes, scale)
    scores = scores.reshape(B, K_SELECT)
    labels = labels.reshape(B, K_SELECT)
    n_idx = n_idx.reshape(B, K_SELECT)
    boxes = boxes.reshape(B, K_SELECT, 4)

    text_masks = jax.nn.sigmoid(pred_logits) > 0.0
    idxV = jnp.broadcast_to(n_idx[:, :, None], (B, K_SELECT, 256))
    text_masks = jnp.take_along_axis(text_masks, idxV, axis=1)

    return (text_masks, scores, labels, boxes)
